# Initial kernel scaffold; baseline (speedup 1.0000x reference)
import functools

import jax
import jax.numpy as jnp
from jax import lax
from jax.experimental import pallas as pl
from jax.experimental.pallas import tpu as pltpu
from jax.experimental.pallas import tpu_sc as plsc

N = 1000
E = 500
D = 64
L = 16
EP = 512
NG = EP // L
NCH = D // L
DEGP = 1008


def _rsqrt16(d):
    i = plsc.bitcast(d, jnp.int32)
    i = jnp.int32(0x5F3759DF) - lax.shift_right_logical(i, 1)
    y = plsc.bitcast(i, jnp.float32)
    for _ in range(3):
        y = y * (1.5 - 0.5 * d * y * y)
    return y


def _build(interpret=False):
    mesh = plsc.VectorSubcoreMesh(
        core_axis_name="c", subcore_axis_name="s", num_cores=2, num_subcores=16
    )

    @functools.partial(
        pl.kernel,
        out_type=jax.ShapeDtypeStruct((EP,), jnp.float32),
        mesh=mesh,
        scratch_types=[
            pltpu.VMEM((N * L,), jnp.float32),
            pltpu.VMEM((N * L,), jnp.float32),
            pltpu.VMEM((N * L,), jnp.float32),
            pltpu.VMEM((N * L,), jnp.float32),
            pltpu.VMEM((DEGP,), jnp.float32),
            pltpu.VMEM((EP,), jnp.float32),
            pltpu.VMEM((EP,), jnp.int32),
            pltpu.VMEM((EP,), jnp.int32),
            pltpu.VMEM((L,), jnp.float32),
            pltpu.VMEM((EP,), jnp.float32),
            pltpu.VMEM((EP,), jnp.float32),
            pltpu.VMEM_SHARED((NCH, EP), jnp.float32),
        ],
        interpret=interpret,
    )
    def gcn_kernel(row_hbm, col_hbm, wc_hbm, alpha_hbm, out_hbm,
                   x0, x1, x2, x3, deg, ew, row, col, alv, part, tmp, shared):
        cid = lax.axis_index("c")
        sid = lax.axis_index("s")
        active = jnp.logical_and(cid == 0, sid < NCH)

        @pl.when(active)
        def _work():
            chunk = sid
            pltpu.sync_copy(row_hbm, row)
            pltpu.sync_copy(col_hbm, col)
            pltpu.sync_copy(wc_hbm.at[chunk], x0)
            pltpu.sync_copy(alpha_hbm, alv)

            zero16 = jnp.zeros((L,), jnp.float32)
            iota = lax.iota(jnp.int32, L)

            def zero_deg(i, c):
                deg[pl.ds(i * L, L)] = zero16
                return c
            lax.fori_loop(0, DEGP // L, zero_deg, 0)

            def deg_scatter(g, c):
                cv = col[pl.ds(g * L, L)]
                valid = jnp.where(g * L + iota < E, 1.0, 0.0)
                plsc.addupdate_scatter(deg, [cv], valid)
                return c
            lax.fori_loop(0, NG, deg_scatter, 0)

            def inv_sqrt(i, c):
                d = deg[pl.ds(i * L, L)]
                y = _rsqrt16(d)
                deg[pl.ds(i * L, L)] = jnp.where(d > 0.0, y, 0.0)
                return c
            lax.fori_loop(0, DEGP // L, inv_sqrt, 0)

            def edge_w(g, c):
                rv = row[pl.ds(g * L, L)]
                cv = col[pl.ds(g * L, L)]
                a = plsc.load_gather(deg, [rv])
                b = plsc.load_gather(deg, [cv])
                valid = jnp.where(g * L + iota < E, 1.0, 0.0)
                ew[pl.ds(g * L, L)] = a * b * valid
                return c
            lax.fori_loop(0, NG, edge_w, 0)

            def zero_x(i, c):
                x1[pl.ds(i * L, L)] = zero16
                x2[pl.ds(i * L, L)] = zero16
                x3[pl.ds(i * L, L)] = zero16
                return c
            lax.fori_loop(0, N, zero_x, 0)

            for src, dst in ((x0, x1), (x1, x2), (x2, x3)):
                def layer(g, c, src=src, dst=dst):
                    rv16 = row[pl.ds(g * L, L)] * L
                    cv16 = col[pl.ds(g * L, L)] * L
                    ewg = ew[pl.ds(g * L, L)]
                    for d in range(L):
                        msg = plsc.load_gather(src, [rv16 + d]) * ewg
                        plsc.addupdate_scatter(dst, [cv16 + d], msg)
                    return c
                lax.fori_loop(0, NG, layer, 0)

            a0 = plsc.load_gather(alv, [jnp.zeros((L,), jnp.int32)])
            a1 = plsc.load_gather(alv, [jnp.full((L,), 1, jnp.int32)])
            a2 = plsc.load_gather(alv, [jnp.full((L,), 2, jnp.int32)])
            a3 = plsc.load_gather(alv, [jnp.full((L,), 3, jnp.int32)])

            def dot(g, c):
                rv16 = row[pl.ds(g * L, L)] * L
                cv16 = col[pl.ds(g * L, L)] * L
                acc = zero16
                for d in range(L):
                    orv = (a0 * plsc.load_gather(x0, [rv16 + d])
                           + a1 * plsc.load_gather(x1, [rv16 + d])
                           + a2 * plsc.load_gather(x2, [rv16 + d])
                           + a3 * plsc.load_gather(x3, [rv16 + d]))
                    ocv = (a0 * plsc.load_gather(x0, [cv16 + d])
                           + a1 * plsc.load_gather(x1, [cv16 + d])
                           + a2 * plsc.load_gather(x2, [cv16 + d])
                           + a3 * plsc.load_gather(x3, [cv16 + d]))
                    acc = acc + orv * ocv
                part[pl.ds(g * L, L)] = acc
                return c
            lax.fori_loop(0, NG, dot, 0)

            pltpu.sync_copy(part, shared.at[chunk])

        plsc.subcore_barrier()

        @pl.when(jnp.logical_and(cid == 0, sid == 0))
        def _reduce():
            for t in range(1, NCH):
                pltpu.sync_copy(shared.at[t], tmp)

                def accum(g, c, t=t):
                    part[pl.ds(g * L, L)] = (part[pl.ds(g * L, L)]
                                             + tmp[pl.ds(g * L, L)])
                    return c
                lax.fori_loop(0, NG, accum, 0)
            pltpu.sync_copy(part, out_hbm)

    return gcn_kernel


_gcn = _build()


def kernel(L_edge_index_, L_self_modules_embedding_parameters_weight_,
           L_self_buffers_alpha_):
    ei = L_edge_index_
    w = L_self_modules_embedding_parameters_weight_
    alpha = L_self_buffers_alpha_
    row_p = jnp.pad(ei[0].astype(jnp.int32), (0, EP - E))
    col_p = jnp.pad(ei[1].astype(jnp.int32), (0, EP - E))
    wc = w.reshape(N, NCH, L).transpose(1, 0, 2).reshape(NCH, N * L)
    alpha_p = jnp.pad(alpha.astype(jnp.float32), (0, L - 4))
    res = _gcn(row_p, col_p, wc, alpha_p)
    return (res[:E],)


# speedup vs baseline: 1.7762x; 1.7762x over previous
import functools

import jax
import jax.numpy as jnp
from jax import lax
from jax.experimental import pallas as pl
from jax.experimental.pallas import tpu as pltpu
from jax.experimental.pallas import tpu_sc as plsc

N = 1000
E = 500
D = 64
L = 16
EP = 512
NG = EP // L
NCH = D // L
DEGP = 1008


def _rsqrt16(d):
    i = plsc.bitcast(d, jnp.int32)
    i = jnp.int32(0x5F3759DF) - lax.shift_right_logical(i, 1)
    y = plsc.bitcast(i, jnp.float32)
    for _ in range(3):
        y = y * (1.5 - 0.5 * d * y * y)
    return y


def _build(interpret=False):
    mesh = plsc.VectorSubcoreMesh(
        core_axis_name="c", subcore_axis_name="s", num_cores=2, num_subcores=16
    )

    @functools.partial(
        pl.kernel,
        out_type=jax.ShapeDtypeStruct((EP,), jnp.float32),
        mesh=mesh,
        scratch_types=[
            pltpu.VMEM((N * L,), jnp.float32),
            pltpu.VMEM((N * L,), jnp.float32),
            pltpu.VMEM((N * L,), jnp.float32),
            pltpu.VMEM((N * L,), jnp.float32),
            pltpu.VMEM((DEGP,), jnp.float32),
            pltpu.VMEM((EP,), jnp.float32),
            pltpu.VMEM((EP,), jnp.int32),
            pltpu.VMEM((EP,), jnp.int32),
            pltpu.VMEM((4 * L,), jnp.float32),
            pltpu.VMEM((EP,), jnp.float32),
            pltpu.VMEM((EP,), jnp.float32),
            pltpu.VMEM_SHARED((NCH, EP), jnp.float32),
        ],
        compiler_params=pltpu.CompilerParams(needs_layout_passes=False),
        interpret=interpret,
    )
    def gcn_kernel(row_hbm, col_hbm, wc_hbm, alpha_hbm, out_hbm,
                   x0, x1, x2, x3, deg, ew, row, col, alv, part, tmp, shared):
        cid = lax.axis_index("c")
        sid = lax.axis_index("s")
        active = jnp.logical_and(cid == 0, sid < NCH)

        @pl.when(active)
        def _work():
            chunk = sid
            pltpu.sync_copy(row_hbm, row)
            pltpu.sync_copy(col_hbm, col)
            pltpu.sync_copy(wc_hbm.at[chunk], x0)
            pltpu.sync_copy(alpha_hbm, alv)

            zero16 = jnp.zeros((L,), jnp.float32)
            iota = lax.iota(jnp.int32, L)

            def zero_deg(i, c):
                deg[pl.ds(i * L, L)] = zero16
                return c
            lax.fori_loop(0, DEGP // L, zero_deg, 0)

            def deg_scatter(g, c):
                cv = col[pl.ds(g * L, L)]
                valid = jnp.where(g * L + iota < E, 1.0, 0.0)
                plsc.addupdate_scatter(deg, [cv], valid)
                return c
            lax.fori_loop(0, NG, deg_scatter, 0)

            def inv_sqrt(i, c):
                d = deg[pl.ds(i * L, L)]
                y = _rsqrt16(d)
                deg[pl.ds(i * L, L)] = jnp.where(d > 0.0, y, 0.0)
                return c
            lax.fori_loop(0, DEGP // L, inv_sqrt, 0)

            def edge_w(g, c):
                rv = row[pl.ds(g * L, L)]
                cv = col[pl.ds(g * L, L)]
                a = plsc.load_gather(deg, [rv])
                b = plsc.load_gather(deg, [cv])
                valid = jnp.where(g * L + iota < E, 1.0, 0.0)
                ew[pl.ds(g * L, L)] = a * b * valid
                return c
            lax.fori_loop(0, NG, edge_w, 0)

            def zero_x(i, c):
                x1[pl.ds(i * L, L)] = zero16
                x2[pl.ds(i * L, L)] = zero16
                x3[pl.ds(i * L, L)] = zero16
                return c
            lax.fori_loop(0, N, zero_x, 0)

            for src, dst in ((x0, x1), (x1, x2), (x2, x3)):
                def layer(g, c, src=src, dst=dst):
                    rv16 = row[pl.ds(g * L, L)] * L
                    cv16 = col[pl.ds(g * L, L)] * L
                    ewg = ew[pl.ds(g * L, L)]
                    for d in range(L):
                        msg = plsc.load_gather(src, [rv16 + d]) * ewg
                        plsc.addupdate_scatter(dst, [cv16 + d], msg)
                    return c
                lax.fori_loop(0, NG, layer, 0)

            a0 = alv[pl.ds(0, L)]
            a1 = alv[pl.ds(L, L)]
            a2 = alv[pl.ds(2 * L, L)]
            a3 = alv[pl.ds(3 * L, L)]

            def dot(g, c):
                rv16 = row[pl.ds(g * L, L)] * L
                cv16 = col[pl.ds(g * L, L)] * L
                acc = zero16
                for d in range(L):
                    orv = (a0 * plsc.load_gather(x0, [rv16 + d])
                           + a1 * plsc.load_gather(x1, [rv16 + d])
                           + a2 * plsc.load_gather(x2, [rv16 + d])
                           + a3 * plsc.load_gather(x3, [rv16 + d]))
                    ocv = (a0 * plsc.load_gather(x0, [cv16 + d])
                           + a1 * plsc.load_gather(x1, [cv16 + d])
                           + a2 * plsc.load_gather(x2, [cv16 + d])
                           + a3 * plsc.load_gather(x3, [cv16 + d]))
                    acc = acc + orv * ocv
                part[pl.ds(g * L, L)] = acc
                return c
            lax.fori_loop(0, NG, dot, 0)

            pltpu.sync_copy(part, shared.at[chunk])

        plsc.subcore_barrier()

        @pl.when(jnp.logical_and(cid == 0, sid == 0))
        def _reduce():
            for t in range(1, NCH):
                pltpu.sync_copy(shared.at[t], tmp)

                def accum(g, c, t=t):
                    part[pl.ds(g * L, L)] = (part[pl.ds(g * L, L)]
                                             + tmp[pl.ds(g * L, L)])
                    return c
                lax.fori_loop(0, NG, accum, 0)
            pltpu.sync_copy(part, out_hbm)

    return gcn_kernel


_gcn_cache = []


def _gcn(*args):
    if not _gcn_cache:
        _gcn_cache.append(_build())
    return _gcn_cache[0](*args)


def kernel(L_edge_index_, L_self_modules_embedding_parameters_weight_,
           L_self_buffers_alpha_):
    ei = L_edge_index_
    w = L_self_modules_embedding_parameters_weight_
    alpha = L_self_buffers_alpha_
    row_p = jnp.pad(ei[0].astype(jnp.int32), (0, EP - E))
    col_p = jnp.pad(ei[1].astype(jnp.int32), (0, EP - E))
    wc = w.reshape(N, NCH, L).transpose(1, 0, 2).reshape(NCH, N * L)
    alpha_p = jnp.tile(alpha.astype(jnp.float32)[:, None], (1, L)).reshape(4 * L)
    res = _gcn(row_p, col_p, wc, alpha_p)
    return (res[:E],)


# speedup vs baseline: 2.1292x; 1.1988x over previous
import functools

import jax
import jax.numpy as jnp
from jax import lax
from jax.experimental import pallas as pl
from jax.experimental.pallas import tpu as pltpu
from jax.experimental.pallas import tpu_sc as plsc

N = 1000
E = 500
D = 64
L = 16
EP = 512
NG = EP // L
NCH = D // L
DEGP = 1008


def _rsqrt16(d):
    i = plsc.bitcast(d, jnp.int32)
    i = jnp.int32(0x5F3759DF) - lax.shift_right_logical(i, 1)
    y = plsc.bitcast(i, jnp.float32)
    for _ in range(3):
        y = y * (1.5 - 0.5 * d * y * y)
    return y


def _build(interpret=False):
    mesh = plsc.VectorSubcoreMesh(
        core_axis_name="c", subcore_axis_name="s", num_cores=2, num_subcores=16
    )

    @functools.partial(
        pl.kernel,
        out_type=jax.ShapeDtypeStruct((EP,), jnp.float32),
        mesh=mesh,
        scratch_types=[
            pltpu.VMEM((N * L,), jnp.float32),
            pltpu.VMEM((N * L,), jnp.float32),
            pltpu.VMEM((N * L,), jnp.float32),
            pltpu.VMEM((N * L,), jnp.float32),
            pltpu.VMEM((DEGP,), jnp.float32),
            pltpu.VMEM((EP,), jnp.float32),
            pltpu.VMEM((EP,), jnp.int32),
            pltpu.VMEM((EP,), jnp.int32),
            pltpu.VMEM((4 * L,), jnp.float32),
            pltpu.VMEM((EP,), jnp.float32),
            pltpu.VMEM((EP,), jnp.float32),
            pltpu.VMEM_SHARED((NCH, EP), jnp.float32),
        ],
        compiler_params=pltpu.CompilerParams(needs_layout_passes=False),
        interpret=interpret,
    )
    def gcn_kernel(row_hbm, col_hbm, wc_hbm, alpha_hbm, out_hbm,
                   x0, x1, x2, x3, deg, ew, row, col, alv, part, tmp, shared):
        cid = lax.axis_index("c")
        sid = lax.axis_index("s")
        active = jnp.logical_and(cid == 0, sid < NCH)

        @pl.when(active)
        def _work():
            chunk = sid
            pltpu.sync_copy(row_hbm, row)
            pltpu.sync_copy(col_hbm, col)
            pltpu.sync_copy(wc_hbm.at[chunk], x0)
            pltpu.sync_copy(alpha_hbm, alv)

            zero16 = jnp.zeros((L,), jnp.float32)
            iota = lax.iota(jnp.int32, L)

            @plsc.parallel_loop(0, DEGP // L, unroll=4)
            def zero_deg(i):
                deg[pl.ds(i * L, L)] = zero16

            @plsc.parallel_loop(0, NG, unroll=2)
            def deg_scatter(g):
                cv = col[pl.ds(g * L, L)]
                valid = jnp.where(g * L + iota < E, 1.0, 0.0)
                plsc.addupdate_scatter(deg, [cv], valid)

            @plsc.parallel_loop(0, DEGP // L, unroll=2)
            def inv_sqrt(i):
                d = deg[pl.ds(i * L, L)]
                y = _rsqrt16(d)
                deg[pl.ds(i * L, L)] = jnp.where(d > 0.0, y, 0.0)

            @plsc.parallel_loop(0, NG, unroll=2)
            def edge_w(g):
                rv = row[pl.ds(g * L, L)]
                cv = col[pl.ds(g * L, L)]
                a = plsc.load_gather(deg, [rv])
                b = plsc.load_gather(deg, [cv])
                valid = jnp.where(g * L + iota < E, 1.0, 0.0)
                ew[pl.ds(g * L, L)] = a * b * valid

            @plsc.parallel_loop(0, N, unroll=8)
            def zero_x(i):
                x1[pl.ds(i * L, L)] = zero16
                x2[pl.ds(i * L, L)] = zero16
                x3[pl.ds(i * L, L)] = zero16

            for src, dst in ((x0, x1), (x1, x2), (x2, x3)):
                @plsc.parallel_loop(0, NG)
                def layer(g, src=src, dst=dst):
                    rv16 = row[pl.ds(g * L, L)] * L
                    cv16 = col[pl.ds(g * L, L)] * L
                    ewg = ew[pl.ds(g * L, L)]
                    for d in range(L):
                        msg = plsc.load_gather(src, [rv16 + d]) * ewg
                        plsc.addupdate_scatter(dst, [cv16 + d], msg)

            a0 = alv[pl.ds(0, L)]
            a1 = alv[pl.ds(L, L)]
            a2 = alv[pl.ds(2 * L, L)]
            a3 = alv[pl.ds(3 * L, L)]

            @plsc.parallel_loop(0, NG)
            def dot(g):
                rv16 = row[pl.ds(g * L, L)] * L
                cv16 = col[pl.ds(g * L, L)] * L
                acc = zero16
                for d in range(L):
                    orv = (a0 * plsc.load_gather(x0, [rv16 + d])
                           + a1 * plsc.load_gather(x1, [rv16 + d])
                           + a2 * plsc.load_gather(x2, [rv16 + d])
                           + a3 * plsc.load_gather(x3, [rv16 + d]))
                    ocv = (a0 * plsc.load_gather(x0, [cv16 + d])
                           + a1 * plsc.load_gather(x1, [cv16 + d])
                           + a2 * plsc.load_gather(x2, [cv16 + d])
                           + a3 * plsc.load_gather(x3, [cv16 + d]))
                    acc = acc + orv * ocv
                part[pl.ds(g * L, L)] = acc

            pltpu.sync_copy(part, shared.at[chunk])

        plsc.subcore_barrier()

        @pl.when(jnp.logical_and(cid == 0, sid == 0))
        def _reduce():
            for t in range(1, NCH):
                pltpu.sync_copy(shared.at[t], tmp)

                @plsc.parallel_loop(0, NG, unroll=2)
                def accum(g, t=t):
                    part[pl.ds(g * L, L)] = (part[pl.ds(g * L, L)]
                                             + tmp[pl.ds(g * L, L)])
            pltpu.sync_copy(part, out_hbm)

    return gcn_kernel


_gcn_cache = []


def _gcn(*args):
    if not _gcn_cache:
        _gcn_cache.append(_build())
    return _gcn_cache[0](*args)


def kernel(L_edge_index_, L_self_modules_embedding_parameters_weight_,
           L_self_buffers_alpha_):
    ei = L_edge_index_
    w = L_self_modules_embedding_parameters_weight_
    alpha = L_self_buffers_alpha_
    row_p = jnp.pad(ei[0].astype(jnp.int32), (0, EP - E))
    col_p = jnp.pad(ei[1].astype(jnp.int32), (0, EP - E))
    wc = w.reshape(N, NCH, L).transpose(1, 0, 2).reshape(NCH, N * L)
    alpha_p = jnp.tile(alpha.astype(jnp.float32)[:, None], (1, L)).reshape(4 * L)
    res = _gcn(row_p, col_p, wc, alpha_p)
    return (res[:E],)


# speedup vs baseline: 2.1684x; 1.0184x over previous
import functools

import jax
import jax.numpy as jnp
from jax import lax
from jax.experimental import pallas as pl
from jax.experimental.pallas import tpu as pltpu
from jax.experimental.pallas import tpu_sc as plsc

N = 1000
E = 500
D = 64
L = 16
EP = 512
NG = EP // L
NCH = D // L
DEGP = 1008


def _rsqrt16(d):
    i = plsc.bitcast(d, jnp.int32)
    i = jnp.int32(0x5F3759DF) - lax.shift_right_logical(i, 1)
    y = plsc.bitcast(i, jnp.float32)
    for _ in range(3):
        y = y * (1.5 - 0.5 * d * y * y)
    return y


def _build(interpret=False):
    mesh = plsc.VectorSubcoreMesh(
        core_axis_name="c", subcore_axis_name="s", num_cores=2, num_subcores=16
    )

    @functools.partial(
        pl.kernel,
        out_type=jax.ShapeDtypeStruct((EP,), jnp.float32),
        mesh=mesh,
        scratch_types=[
            pltpu.VMEM((N * L,), jnp.float32),
            pltpu.VMEM((N * L,), jnp.float32),
            pltpu.VMEM((N * L,), jnp.float32),
            pltpu.VMEM((N * L,), jnp.float32),
            pltpu.VMEM((DEGP,), jnp.float32),
            pltpu.VMEM((EP,), jnp.float32),
            pltpu.VMEM((EP,), jnp.int32),
            pltpu.VMEM((EP,), jnp.int32),
            pltpu.VMEM((4 * L,), jnp.float32),
            pltpu.VMEM((EP,), jnp.float32),
            pltpu.VMEM((EP,), jnp.float32),
            pltpu.VMEM_SHARED((NCH, EP), jnp.float32),
        ],
        compiler_params=pltpu.CompilerParams(needs_layout_passes=False),
        interpret=interpret,
    )
    def gcn_kernel(row_hbm, col_hbm, wc_hbm, alpha_hbm, out_hbm,
                   x0, x1, x2, x3, deg, ew, row, col, alv, part, tmp, shared):
        cid = lax.axis_index("c")
        sid = lax.axis_index("s")
        active = jnp.logical_and(cid == 0, sid < NCH)

        @pl.when(active)
        def _work():
            chunk = sid
            pltpu.sync_copy(row_hbm, row)
            pltpu.sync_copy(col_hbm, col)
            pltpu.sync_copy(wc_hbm.at[chunk], x0)
            pltpu.sync_copy(alpha_hbm, alv)

            zero16 = jnp.zeros((L,), jnp.float32)
            iota = lax.iota(jnp.int32, L)

            @plsc.parallel_loop(0, DEGP // L, unroll=4)
            def zero_deg(i):
                deg[pl.ds(i * L, L)] = zero16

            @plsc.parallel_loop(0, NG, unroll=2)
            def deg_scatter(g):
                cv = col[pl.ds(g * L, L)]
                valid = jnp.where(g * L + iota < E, 1.0, 0.0)
                plsc.addupdate_scatter(deg, [cv], valid)

            @plsc.parallel_loop(0, DEGP // L, unroll=2)
            def inv_sqrt(i):
                d = deg[pl.ds(i * L, L)]
                y = _rsqrt16(d)
                deg[pl.ds(i * L, L)] = jnp.where(d > 0.0, y, 0.0)

            @plsc.parallel_loop(0, NG, unroll=2)
            def edge_w(g):
                rv = row[pl.ds(g * L, L)]
                cv = col[pl.ds(g * L, L)]
                a = plsc.load_gather(deg, [rv])
                b = plsc.load_gather(deg, [cv])
                valid = jnp.where(g * L + iota < E, 1.0, 0.0)
                ew[pl.ds(g * L, L)] = a * b * valid

            @plsc.parallel_loop(0, N, unroll=8)
            def zero_x(i):
                x1[pl.ds(i * L, L)] = zero16
                x2[pl.ds(i * L, L)] = zero16
                x3[pl.ds(i * L, L)] = zero16

            for src, dst in ((x0, x1), (x1, x2), (x2, x3)):
                @plsc.parallel_loop(0, NG, unroll=2)
                def layer(g, src=src, dst=dst):
                    rv16 = row[pl.ds(g * L, L)] * L
                    cv16 = col[pl.ds(g * L, L)] * L
                    ewg = ew[pl.ds(g * L, L)]
                    for d in range(L):
                        msg = plsc.load_gather(src, [rv16 + d]) * ewg
                        plsc.addupdate_scatter(dst, [cv16 + d], msg)

            a0 = alv[pl.ds(0, L)]
            a1 = alv[pl.ds(L, L)]
            a2 = alv[pl.ds(2 * L, L)]
            a3 = alv[pl.ds(3 * L, L)]

            @plsc.parallel_loop(0, NG, unroll=2)
            def dot(g):
                rv16 = row[pl.ds(g * L, L)] * L
                cv16 = col[pl.ds(g * L, L)] * L
                acc = zero16
                for d in range(L):
                    orv = (a0 * plsc.load_gather(x0, [rv16 + d])
                           + a1 * plsc.load_gather(x1, [rv16 + d])
                           + a2 * plsc.load_gather(x2, [rv16 + d])
                           + a3 * plsc.load_gather(x3, [rv16 + d]))
                    ocv = (a0 * plsc.load_gather(x0, [cv16 + d])
                           + a1 * plsc.load_gather(x1, [cv16 + d])
                           + a2 * plsc.load_gather(x2, [cv16 + d])
                           + a3 * plsc.load_gather(x3, [cv16 + d]))
                    acc = acc + orv * ocv
                part[pl.ds(g * L, L)] = acc

            pltpu.sync_copy(part, shared.at[chunk])

        plsc.subcore_barrier()

        @pl.when(jnp.logical_and(cid == 0, sid == 0))
        def _reduce():
            for t in range(1, NCH):
                pltpu.sync_copy(shared.at[t], tmp)

                @plsc.parallel_loop(0, NG, unroll=2)
                def accum(g, t=t):
                    part[pl.ds(g * L, L)] = (part[pl.ds(g * L, L)]
                                             + tmp[pl.ds(g * L, L)])
            pltpu.sync_copy(part, out_hbm)

    return gcn_kernel


_gcn_cache = []


def _gcn(*args):
    if not _gcn_cache:
        _gcn_cache.append(_build())
    return _gcn_cache[0](*args)


def kernel(L_edge_index_, L_self_modules_embedding_parameters_weight_,
           L_self_buffers_alpha_):
    ei = L_edge_index_
    w = L_self_modules_embedding_parameters_weight_
    alpha = L_self_buffers_alpha_
    row_p = jnp.pad(ei[0].astype(jnp.int32), (0, EP - E))
    col_p = jnp.pad(ei[1].astype(jnp.int32), (0, EP - E))
    wc = w.reshape(N, NCH, L).transpose(1, 0, 2).reshape(NCH, N * L)
    alpha_p = jnp.tile(alpha.astype(jnp.float32)[:, None], (1, L)).reshape(4 * L)
    res = _gcn(row_p, col_p, wc, alpha_p)
    return (res[:E],)


# speedup vs baseline: 2.4746x; 1.1413x over previous
import functools

import jax
import jax.numpy as jnp
from jax import lax
from jax.experimental import pallas as pl
from jax.experimental.pallas import tpu as pltpu
from jax.experimental.pallas import tpu_sc as plsc

N = 1000
E = 500
D = 64
L = 16
EP = 512
NB = 4
EB = 128
NCH = D // L
DEGP = 1008


def _rsqrt16(d):
    i = plsc.bitcast(d, jnp.int32)
    i = jnp.int32(0x5F3759DF) - lax.shift_right_logical(i, 1)
    y = plsc.bitcast(i, jnp.float32)
    for _ in range(3):
        y = y * (1.5 - 0.5 * d * y * y)
    return y


def _build(interpret=False):
    mesh = plsc.VectorSubcoreMesh(
        core_axis_name="c", subcore_axis_name="s", num_cores=2, num_subcores=16
    )

    @functools.partial(
        pl.kernel,
        out_type=(jax.ShapeDtypeStruct((EP,), jnp.float32),
                  jax.ShapeDtypeStruct((NCH, N, L), jnp.float32),
                  jax.ShapeDtypeStruct((NCH, N, L), jnp.float32)),
        mesh=mesh,
        scratch_types=[
            pltpu.VMEM((N, L), jnp.float32),
            pltpu.VMEM((N, L), jnp.float32),
            pltpu.VMEM((N, L), jnp.float32),
            pltpu.VMEM((N, L), jnp.float32),
            pltpu.VMEM((N * L,), jnp.float32),
            pltpu.VMEM((NB, EB, L), jnp.float32),
            pltpu.VMEM((NB, EB, L), jnp.float32),
            pltpu.VMEM((DEGP,), jnp.float32),
            pltpu.VMEM((NB, EB), jnp.float32),
            pltpu.VMEM((NB, EB), jnp.int32),
            pltpu.VMEM((NB, EB), jnp.int32),
            pltpu.VMEM((4 * L,), jnp.float32),
            pltpu.VMEM((EP,), jnp.float32),
            pltpu.VMEM((EP,), jnp.float32),
            pltpu.VMEM_SHARED((NCH, N, L), jnp.float32),
            pltpu.VMEM_SHARED((NCH, EP), jnp.float32),
        ],
        compiler_params=pltpu.CompilerParams(needs_layout_passes=False, use_tc_tiling_on_sc=False),
        interpret=interpret,
    )
    def gcn_kernel(ei_hbm, wc_hbm, alpha_hbm, out_hbm, h1_hbm, h2_hbm,
                   x0v, s1v, s2v, s3v, out_f, rows, ewb,
                   deg, ew, rowi, coli, alv, part, tmp, xsp, shared):
        cid = lax.axis_index("c")
        sid = lax.axis_index("s")
        active = jnp.logical_and(cid == 0, sid < NCH)

        @pl.when(active)
        def _work():
            chunk = sid
            pltpu.sync_copy(ei_hbm.at[0], rowi)
            pltpu.sync_copy(ei_hbm.at[1], coli)
            pltpu.sync_copy(wc_hbm.at[chunk], x0v)
            pltpu.sync_copy(alpha_hbm, alv)

            zero16 = jnp.zeros((L,), jnp.float32)
            iota = lax.iota(jnp.int32, L)

            @plsc.parallel_loop(0, N, unroll=4)
            def zero_s(i):
                s1v[i, :] = zero16
                s2v[i, :] = zero16
                s3v[i, :] = zero16

            pltpu.sync_copy(s1v, xsp.at[chunk])

            @plsc.parallel_loop(0, DEGP // L, unroll=4)
            def zero_deg(i):
                deg[pl.ds(i * L, L)] = zero16

            for j in range(NB):
                @plsc.parallel_loop(0, EB // L, unroll=2)
                def deg_scatter(o, j=j):
                    cv = coli[j, pl.ds(o * L, L)]
                    valid = jnp.where(j * EB + o * L + iota < E, 1.0, 0.0)
                    plsc.addupdate_scatter(deg, [cv], valid)

            @plsc.parallel_loop(0, DEGP // L, unroll=2)
            def inv_sqrt(i):
                d = deg[pl.ds(i * L, L)]
                y = _rsqrt16(d)
                deg[pl.ds(i * L, L)] = jnp.where(d > 0.0, y, 0.0)

            for j in range(NB):
                @plsc.parallel_loop(0, EB // L, unroll=2)
                def edge_w(o, j=j):
                    rv = rowi[j, pl.ds(o * L, L)]
                    cv = coli[j, pl.ds(o * L, L)]
                    a = plsc.load_gather(deg, [rv])
                    b = plsc.load_gather(deg, [cv])
                    valid = jnp.where(j * EB + o * L + iota < E, 1.0, 0.0)
                    ew[j, pl.ds(o * L, L)] = a * b * valid

            for j in range(NB):
                @plsc.parallel_loop(0, EB // L)
                def bcast(o, j=j):
                    ewg = ew[j, pl.ds(o * L, L)]
                    for t in range(L):
                        ewb[j, o * L + t, :] = jnp.broadcast_to(ewg[t], (L,))

            for src_hbm, skv, hkv, zsv in (
                    (wc_hbm, s1v, h1_hbm, s2v),
                    (h1_hbm, s2v, h2_hbm, s3v),
                    (h2_hbm, s3v, None, None)):
                for j in range(NB):
                    pltpu.sync_copy(src_hbm.at[chunk].at[rowi.at[j]],
                                    rows.at[j])

                for j in range(NB):
                    @plsc.parallel_loop(0, EB, unroll=2)
                    def scale(e, j=j):
                        rows[j, e, :] = rows[j, e, :] * ewb[j, e, :]

                for j in range(NB):
                    pltpu.sync_copy(rows.at[j],
                                    xsp.at[chunk].at[coli.at[j]],
                                    add=True)

                pltpu.sync_copy(xsp.at[chunk], skv)
                if hkv is not None:
                    pltpu.sync_copy(skv, hkv.at[chunk])
                if zsv is not None:
                    pltpu.sync_copy(zsv, xsp.at[chunk])

            a0 = alv[pl.ds(0, L)]
            a1 = alv[pl.ds(L, L)]
            a2 = alv[pl.ds(2 * L, L)]
            a3 = alv[pl.ds(3 * L, L)]

            @plsc.parallel_loop(0, N, unroll=2)
            def combine(i):
                out_f[pl.ds(i * L, L)] = (a0 * x0v[i, :] + a1 * s1v[i, :]
                                          + a2 * s2v[i, :] + a3 * s3v[i, :])

            for j in range(NB):
                @plsc.parallel_loop(0, EB // L)
                def dot(o, j=j):
                    rv16 = rowi[j, pl.ds(o * L, L)] * L
                    cv16 = coli[j, pl.ds(o * L, L)] * L
                    acc = zero16
                    for d in range(L):
                        acc = acc + (plsc.load_gather(out_f, [rv16 + d])
                                     * plsc.load_gather(out_f, [cv16 + d]))
                    part[pl.ds(j * EB + o * L, L)] = acc

            pltpu.sync_copy(part, shared.at[chunk])

        plsc.subcore_barrier()

        @pl.when(jnp.logical_and(cid == 0, sid == 0))
        def _reduce():
            for t in range(1, NCH):
                pltpu.sync_copy(shared.at[t], tmp)

                @plsc.parallel_loop(0, EP // L, unroll=2)
                def accum(g, t=t):
                    part[pl.ds(g * L, L)] = (part[pl.ds(g * L, L)]
                                             + tmp[pl.ds(g * L, L)])
            pltpu.sync_copy(part, out_hbm)

    return gcn_kernel


_gcn_cache = []


def _gcn(*args):
    if not _gcn_cache:
        _gcn_cache.append(_build())
    return _gcn_cache[0](*args)


def kernel(L_edge_index_, L_self_modules_embedding_parameters_weight_,
           L_self_buffers_alpha_):
    ei = L_edge_index_
    w = L_self_modules_embedding_parameters_weight_
    alpha = L_self_buffers_alpha_
    ei_p = jnp.pad(ei.astype(jnp.int32), ((0, 0), (0, EP - E))).reshape(
        2, NB, EB)
    wc = w.reshape(N, NCH, L).transpose(1, 0, 2)
    alpha_p = jnp.tile(alpha.astype(jnp.float32)[:, None], (1, L)).reshape(
        4 * L)
    res, _, _ = _gcn(ei_p, wc, alpha_p)
    return (res[:E],)


# speedup vs baseline: 2.8536x; 1.1531x over previous
import functools

import jax
import jax.numpy as jnp
from jax import lax
from jax.experimental import pallas as pl
from jax.experimental.pallas import tpu as pltpu
from jax.experimental.pallas import tpu_sc as plsc

N = 1000
E = 500
D = 64
L = 16
EP = 512
NB = 4
EB = 128
NCH = D // L
DEGP = 1008


def _rsqrt16(d):
    i = plsc.bitcast(d, jnp.int32)
    i = jnp.int32(0x5F3759DF) - lax.shift_right_logical(i, 1)
    y = plsc.bitcast(i, jnp.float32)
    for _ in range(3):
        y = y * (1.5 - 0.5 * d * y * y)
    return y


def _build(interpret=False):
    mesh = plsc.VectorSubcoreMesh(
        core_axis_name="c", subcore_axis_name="s", num_cores=2, num_subcores=16
    )

    @functools.partial(
        pl.kernel,
        out_type=jax.ShapeDtypeStruct((EP,), jnp.float32),
        mesh=mesh,
        scratch_types=[
            pltpu.VMEM((N, L), jnp.float32),
            pltpu.VMEM((N, L), jnp.float32),
            pltpu.VMEM((N, L), jnp.float32),
            pltpu.VMEM((N, L), jnp.float32),
            pltpu.VMEM((N * L,), jnp.float32),
            pltpu.VMEM((NB, EB, L), jnp.float32),
            pltpu.VMEM((NB, EB, L), jnp.float32),
            pltpu.VMEM((DEGP,), jnp.float32),
            pltpu.VMEM((NB, EB), jnp.float32),
            pltpu.VMEM((NB, EB), jnp.int32),
            pltpu.VMEM((NB, EB), jnp.int32),
            pltpu.VMEM((4 * L,), jnp.float32),
            pltpu.VMEM((EP,), jnp.float32),
            pltpu.VMEM((EP,), jnp.float32),
            pltpu.VMEM_SHARED((NCH, 2, N, L), jnp.float32),
            pltpu.VMEM_SHARED((NCH, EP), jnp.float32),
        ],
        compiler_params=pltpu.CompilerParams(needs_layout_passes=False, use_tc_tiling_on_sc=False),
        interpret=interpret,
    )
    def gcn_kernel(ei_hbm, wc_hbm, alpha_hbm, out_hbm,
                   x0v, s1v, s2v, s3v, out_f, rows, ewb,
                   deg, ew, rowi, coli, alv, part, tmp, xsp, shared):
        cid = lax.axis_index("c")
        sid = lax.axis_index("s")
        active = jnp.logical_and(cid == 0, sid < NCH)

        @pl.when(active)
        def _work():
            chunk = sid
            pltpu.sync_copy(ei_hbm.at[0], rowi)
            pltpu.sync_copy(ei_hbm.at[1], coli)
            pltpu.sync_copy(wc_hbm.at[chunk], x0v)
            pltpu.sync_copy(wc_hbm.at[chunk], xsp.at[chunk, 0])
            pltpu.sync_copy(alpha_hbm, alv)

            zero16 = jnp.zeros((L,), jnp.float32)
            iota = lax.iota(jnp.int32, L)

            @plsc.parallel_loop(0, N, unroll=4)
            def zero_s(i):
                s1v[i, :] = zero16
                s2v[i, :] = zero16
                s3v[i, :] = zero16

            pltpu.sync_copy(s1v, xsp.at[chunk, 1])

            @plsc.parallel_loop(0, DEGP // L, unroll=4)
            def zero_deg(i):
                deg[pl.ds(i * L, L)] = zero16

            for j in range(NB):
                @plsc.parallel_loop(0, EB // L, unroll=2)
                def deg_scatter(o, j=j):
                    cv = coli[j, pl.ds(o * L, L)]
                    valid = jnp.where(j * EB + o * L + iota < E, 1.0, 0.0)
                    plsc.addupdate_scatter(deg, [cv], valid)

            @plsc.parallel_loop(0, DEGP // L, unroll=2)
            def inv_sqrt(i):
                d = deg[pl.ds(i * L, L)]
                y = _rsqrt16(d)
                deg[pl.ds(i * L, L)] = jnp.where(d > 0.0, y, 0.0)

            for j in range(NB):
                @plsc.parallel_loop(0, EB // L, unroll=2)
                def edge_w(o, j=j):
                    rv = rowi[j, pl.ds(o * L, L)]
                    cv = coli[j, pl.ds(o * L, L)]
                    a = plsc.load_gather(deg, [rv])
                    b = plsc.load_gather(deg, [cv])
                    valid = jnp.where(j * EB + o * L + iota < E, 1.0, 0.0)
                    ew[j, pl.ds(o * L, L)] = a * b * valid

            for j in range(NB):
                @plsc.parallel_loop(0, EB // L)
                def bcast(o, j=j):
                    ewg = ew[j, pl.ds(o * L, L)]
                    for t in range(L):
                        ewb[j, o * L + t, :] = jnp.broadcast_to(ewg[t], (L,))

            for k, skv, zsv in ((1, s1v, s2v), (2, s2v, s3v), (3, s3v, None)):
                srcr = (k - 1) % 2
                dstr = k % 2
                for j in range(NB):
                    pltpu.sync_copy(xsp.at[chunk, srcr].at[rowi.at[j]],
                                    rows.at[j])

                for j in range(NB):
                    @plsc.parallel_loop(0, EB, unroll=2)
                    def scale(e, j=j):
                        rows[j, e, :] = rows[j, e, :] * ewb[j, e, :]

                for j in range(NB):
                    pltpu.sync_copy(rows.at[j],
                                    xsp.at[chunk, dstr].at[coli.at[j]],
                                    add=True)

                pltpu.sync_copy(xsp.at[chunk, dstr], skv)
                if zsv is not None:
                    pltpu.sync_copy(zsv, xsp.at[chunk, srcr])

            a0 = alv[pl.ds(0, L)]
            a1 = alv[pl.ds(L, L)]
            a2 = alv[pl.ds(2 * L, L)]
            a3 = alv[pl.ds(3 * L, L)]

            @plsc.parallel_loop(0, N, unroll=2)
            def combine(i):
                out_f[pl.ds(i * L, L)] = (a0 * x0v[i, :] + a1 * s1v[i, :]
                                          + a2 * s2v[i, :] + a3 * s3v[i, :])

            for j in range(NB):
                @plsc.parallel_loop(0, EB // L)
                def dot(o, j=j):
                    rv16 = rowi[j, pl.ds(o * L, L)] * L
                    cv16 = coli[j, pl.ds(o * L, L)] * L
                    acc = zero16
                    for d in range(L):
                        acc = acc + (plsc.load_gather(out_f, [rv16 + d])
                                     * plsc.load_gather(out_f, [cv16 + d]))
                    part[pl.ds(j * EB + o * L, L)] = acc

            pltpu.sync_copy(part, shared.at[chunk])

        plsc.subcore_barrier()

        @pl.when(jnp.logical_and(cid == 0, sid == 0))
        def _reduce():
            for t in range(1, NCH):
                pltpu.sync_copy(shared.at[t], tmp)

                @plsc.parallel_loop(0, EP // L, unroll=2)
                def accum(g, t=t):
                    part[pl.ds(g * L, L)] = (part[pl.ds(g * L, L)]
                                             + tmp[pl.ds(g * L, L)])
            pltpu.sync_copy(part, out_hbm)

    return gcn_kernel


_gcn_cache = []


def _gcn(*args):
    if not _gcn_cache:
        _gcn_cache.append(_build())
    return _gcn_cache[0](*args)


def kernel(L_edge_index_, L_self_modules_embedding_parameters_weight_,
           L_self_buffers_alpha_):
    ei = L_edge_index_
    w = L_self_modules_embedding_parameters_weight_
    alpha = L_self_buffers_alpha_
    ei_p = jnp.pad(ei.astype(jnp.int32), ((0, 0), (0, EP - E))).reshape(
        2, NB, EB)
    wc = w.reshape(N, NCH, L).transpose(1, 0, 2)
    alpha_p = jnp.tile(alpha.astype(jnp.float32)[:, None], (1, L)).reshape(
        4 * L)
    res = _gcn(ei_p, wc, alpha_p)
    return (res[:E],)


# speedup vs baseline: 3.1320x; 1.0976x over previous
import functools

import jax
import jax.numpy as jnp
from jax import lax
from jax.experimental import pallas as pl
from jax.experimental.pallas import tpu as pltpu
from jax.experimental.pallas import tpu_sc as plsc

N = 1000
E = 500
D = 64
L = 16
EP = 512
NB = 4
EB = 128
NCH = D // L
DEGP = 1008


def _rsqrt16(d):
    i = plsc.bitcast(d, jnp.int32)
    i = jnp.int32(0x5F3759DF) - lax.shift_right_logical(i, 1)
    y = plsc.bitcast(i, jnp.float32)
    for _ in range(3):
        y = y * (1.5 - 0.5 * d * y * y)
    return y


def _build(interpret=False):
    mesh = plsc.VectorSubcoreMesh(
        core_axis_name="c", subcore_axis_name="s", num_cores=2, num_subcores=16
    )

    @functools.partial(
        pl.kernel,
        out_type=jax.ShapeDtypeStruct((EP,), jnp.float32),
        mesh=mesh,
        scratch_types=[
            pltpu.VMEM((N, L), jnp.float32),
            pltpu.VMEM((N, L), jnp.float32),
            pltpu.VMEM((N, L), jnp.float32),
            pltpu.VMEM((N, L), jnp.float32),
            pltpu.VMEM((N * L,), jnp.float32),
            pltpu.VMEM((EP, L), jnp.float32),
            pltpu.VMEM((EP, L), jnp.float32),
            pltpu.VMEM((DEGP,), jnp.float32),
            pltpu.VMEM((EP,), jnp.float32),
            pltpu.VMEM((EP,), jnp.int32),
            pltpu.VMEM((EP,), jnp.int32),
            pltpu.VMEM((4 * L,), jnp.float32),
            pltpu.VMEM((EP,), jnp.float32),
            pltpu.VMEM((EP,), jnp.float32),
            pltpu.VMEM_SHARED((NCH, 2, N, L), jnp.float32),
            pltpu.VMEM_SHARED((NCH, EP), jnp.float32),
        ],
        compiler_params=pltpu.CompilerParams(needs_layout_passes=False, use_tc_tiling_on_sc=False),
        interpret=interpret,
    )
    def gcn_kernel(ei_hbm, wc_hbm, alpha_hbm, out_hbm,
                   x0v, s1v, s2v, s3v, out_f, rows, ewb,
                   deg, ew, rowi, coli, alv, part, tmp, xsp, shared):
        cid = lax.axis_index("c")
        sid = lax.axis_index("s")
        active = jnp.logical_and(cid == 0, sid < NCH)

        @pl.when(active)
        def _work():
            chunk = sid
            pltpu.sync_copy(ei_hbm.at[0], rowi)
            pltpu.sync_copy(ei_hbm.at[1], coli)
            pltpu.sync_copy(wc_hbm.at[chunk], x0v)
            pltpu.sync_copy(wc_hbm.at[chunk], xsp.at[chunk, 0])
            pltpu.sync_copy(alpha_hbm, alv)

            zero16 = jnp.zeros((L,), jnp.float32)
            iota = lax.iota(jnp.int32, L)

            @plsc.parallel_loop(0, N, unroll=4)
            def zero_s(i):
                s1v[i, :] = zero16
                s2v[i, :] = zero16
                s3v[i, :] = zero16

            pltpu.sync_copy(s1v, xsp.at[chunk, 1])

            @plsc.parallel_loop(0, DEGP // L, unroll=4)
            def zero_deg(i):
                deg[pl.ds(i * L, L)] = zero16

            @plsc.parallel_loop(0, EP // L, unroll=2)
            def deg_scatter(g):
                cv = coli[pl.ds(g * L, L)]
                valid = jnp.where(g * L + iota < E, 1.0, 0.0)
                plsc.addupdate_scatter(deg, [cv], valid)

            @plsc.parallel_loop(0, DEGP // L, unroll=2)
            def inv_sqrt(i):
                d = deg[pl.ds(i * L, L)]
                y = _rsqrt16(d)
                deg[pl.ds(i * L, L)] = jnp.where(d > 0.0, y, 0.0)

            @plsc.parallel_loop(0, EP // L, unroll=2)
            def edge_w(g):
                rv = rowi[pl.ds(g * L, L)]
                cv = coli[pl.ds(g * L, L)]
                a = plsc.load_gather(deg, [rv])
                b = plsc.load_gather(deg, [cv])
                valid = jnp.where(g * L + iota < E, 1.0, 0.0)
                ew[pl.ds(g * L, L)] = a * b * valid

            @plsc.parallel_loop(0, EP // L)
            def bcast(g):
                ewg = ew[pl.ds(g * L, L)]
                for t in range(L):
                    ewb[g * L + t, :] = jnp.broadcast_to(ewg[t], (L,))

            for k, skv, zsv in ((1, s1v, s2v), (2, s2v, s3v), (3, s3v, None)):
                srcr = (k - 1) % 2
                dstr = k % 2
                pltpu.sync_copy(xsp.at[chunk, srcr].at[rowi], rows)

                @plsc.parallel_loop(0, EP, unroll=4)
                def scale(e):
                    rows[e, :] = rows[e, :] * ewb[e, :]

                pltpu.sync_copy(rows, xsp.at[chunk, dstr].at[coli],
                                add=True)

                pltpu.sync_copy(xsp.at[chunk, dstr], skv)
                if zsv is not None:
                    pltpu.sync_copy(zsv, xsp.at[chunk, srcr])

            a0 = alv[pl.ds(0, L)]
            a1 = alv[pl.ds(L, L)]
            a2 = alv[pl.ds(2 * L, L)]
            a3 = alv[pl.ds(3 * L, L)]

            @plsc.parallel_loop(0, N, unroll=2)
            def combine(i):
                out_f[pl.ds(i * L, L)] = (a0 * x0v[i, :] + a1 * s1v[i, :]
                                          + a2 * s2v[i, :] + a3 * s3v[i, :])

            @plsc.parallel_loop(0, EP // L)
            def dot(g):
                rv16 = rowi[pl.ds(g * L, L)] * L
                cv16 = coli[pl.ds(g * L, L)] * L
                acc = zero16
                for d in range(L):
                    acc = acc + (plsc.load_gather(out_f, [rv16 + d])
                                 * plsc.load_gather(out_f, [cv16 + d]))
                part[pl.ds(g * L, L)] = acc

            pltpu.sync_copy(part, shared.at[chunk])

        plsc.subcore_barrier()

        @pl.when(jnp.logical_and(cid == 0, sid == 0))
        def _reduce():
            for t in range(1, NCH):
                pltpu.sync_copy(shared.at[t], tmp)

                @plsc.parallel_loop(0, EP // L, unroll=2)
                def accum(g, t=t):
                    part[pl.ds(g * L, L)] = (part[pl.ds(g * L, L)]
                                             + tmp[pl.ds(g * L, L)])
            pltpu.sync_copy(part, out_hbm)

    return gcn_kernel


_gcn_cache = []


def _gcn(*args):
    if not _gcn_cache:
        _gcn_cache.append(_build())
    return _gcn_cache[0](*args)


def kernel(L_edge_index_, L_self_modules_embedding_parameters_weight_,
           L_self_buffers_alpha_):
    ei = L_edge_index_
    w = L_self_modules_embedding_parameters_weight_
    alpha = L_self_buffers_alpha_
    ei_p = jnp.pad(ei.astype(jnp.int32), ((0, 0), (0, EP - E)))
    wc = w.reshape(N, NCH, L).transpose(1, 0, 2)
    alpha_p = jnp.tile(alpha.astype(jnp.float32)[:, None], (1, L)).reshape(
        4 * L)
    res = _gcn(ei_p, wc, alpha_p)
    return (res[:E],)


# speedup vs baseline: 3.2331x; 1.0323x over previous
import functools

import jax
import jax.numpy as jnp
from jax import lax
from jax.experimental import pallas as pl
from jax.experimental.pallas import tpu as pltpu
from jax.experimental.pallas import tpu_sc as plsc

N = 1000
E = 500
D = 64
L = 16
EP = 512
NB = 4
EB = 128
NCH = D // L
DEGP = 1008


def _rsqrt16(d):
    i = plsc.bitcast(d, jnp.int32)
    i = jnp.int32(0x5F3759DF) - lax.shift_right_logical(i, 1)
    y = plsc.bitcast(i, jnp.float32)
    for _ in range(3):
        y = y * (1.5 - 0.5 * d * y * y)
    return y


def _build(interpret=False):
    mesh = plsc.VectorSubcoreMesh(
        core_axis_name="c", subcore_axis_name="s", num_cores=2, num_subcores=16
    )

    @functools.partial(
        pl.kernel,
        out_type=jax.ShapeDtypeStruct((EP,), jnp.float32),
        mesh=mesh,
        scratch_types=[
            pltpu.VMEM((N, L), jnp.float32),
            pltpu.VMEM((N, L), jnp.float32),
            pltpu.VMEM((N, L), jnp.float32),
            pltpu.VMEM((N, L), jnp.float32),
            pltpu.VMEM((N * L,), jnp.float32),
            pltpu.VMEM((EP, L), jnp.float32),
            pltpu.VMEM((EP, L), jnp.float32),
            pltpu.VMEM((DEGP,), jnp.float32),
            pltpu.VMEM((EP,), jnp.float32),
            pltpu.VMEM((EP,), jnp.int32),
            pltpu.VMEM((EP,), jnp.int32),
            pltpu.VMEM((4 * L,), jnp.float32),
            pltpu.VMEM((EP, L), jnp.float32),
            pltpu.VMEM((EP,), jnp.float32),
            pltpu.VMEM((EP,), jnp.float32),
            pltpu.SemaphoreType.DMA,
            pltpu.SemaphoreType.DMA,
            pltpu.VMEM_SHARED((NCH, 2, N, L), jnp.float32),
            pltpu.VMEM_SHARED((NCH, EP), jnp.float32),
        ],
        compiler_params=pltpu.CompilerParams(needs_layout_passes=False, use_tc_tiling_on_sc=False),
        interpret=interpret,
    )
    def gcn_kernel(ei_hbm, wc_hbm, alpha_hbm, out_hbm,
                   x0v, s1v, s2v, s3v, out_f, rows, ewb,
                   deg, ew, rowi, coli, alv, zv, part, tmp, semS, semZ,
                   xsp, shared):
        cid = lax.axis_index("c")
        sid = lax.axis_index("s")
        active = jnp.logical_and(cid == 0, sid < NCH)

        @pl.when(active)
        def _work():
            chunk = sid
            pltpu.sync_copy(ei_hbm.at[0], rowi)
            pltpu.sync_copy(ei_hbm.at[1], coli)
            pltpu.sync_copy(wc_hbm.at[chunk], x0v)
            pltpu.sync_copy(wc_hbm.at[chunk], xsp.at[chunk, 0])
            pltpu.sync_copy(alpha_hbm, alv)

            zero16 = jnp.zeros((L,), jnp.float32)
            iota = lax.iota(jnp.int32, L)

            @plsc.parallel_loop(0, EP, unroll=8)
            def zero_zv(i):
                zv[i, :] = zero16

            pltpu.sync_copy(zv, xsp.at[chunk, 1].at[pl.ds(0, EP)])
            pltpu.sync_copy(zv.at[pl.ds(0, N - EP)],
                            xsp.at[chunk, 1].at[pl.ds(EP, N - EP)])

            @plsc.parallel_loop(0, DEGP // L, unroll=4)
            def zero_deg(i):
                deg[pl.ds(i * L, L)] = zero16

            @plsc.parallel_loop(0, EP // L, unroll=2)
            def deg_scatter(g):
                cv = coli[pl.ds(g * L, L)]
                valid = jnp.where(g * L + iota < E, 1.0, 0.0)
                plsc.addupdate_scatter(deg, [cv], valid)

            @plsc.parallel_loop(0, DEGP // L, unroll=2)
            def inv_sqrt(i):
                d = deg[pl.ds(i * L, L)]
                y = _rsqrt16(d)
                deg[pl.ds(i * L, L)] = jnp.where(d > 0.0, y, 0.0)

            @plsc.parallel_loop(0, EP // L, unroll=2)
            def edge_w(g):
                rv = rowi[pl.ds(g * L, L)]
                cv = coli[pl.ds(g * L, L)]
                a = plsc.load_gather(deg, [rv])
                b = plsc.load_gather(deg, [cv])
                valid = jnp.where(g * L + iota < E, 1.0, 0.0)
                ew[pl.ds(g * L, L)] = a * b * valid

            @plsc.parallel_loop(0, EP // L)
            def bcast(g):
                ewg = ew[pl.ds(g * L, L)]
                for t in range(L):
                    ewb[g * L + t, :] = jnp.broadcast_to(ewg[t], (L,))

            for k, skv, zsv in ((1, s1v, s2v), (2, s2v, s3v), (3, s3v, None)):
                srcr = (k - 1) % 2
                dstr = k % 2
                pltpu.sync_copy(xsp.at[chunk, srcr].at[rowi], rows)

                @plsc.parallel_loop(0, EP, unroll=4)
                def scale(e):
                    rows[e, :] = rows[e, :] * ewb[e, :]

                pltpu.sync_copy(rows, xsp.at[chunk, dstr].at[coli],
                                add=True)

                pltpu.sync_copy(xsp.at[chunk, dstr], skv)
                if zsv is not None:
                    pltpu.sync_copy(zsv, xsp.at[chunk, srcr])

            a0 = alv[pl.ds(0, L)]
            a1 = alv[pl.ds(L, L)]
            a2 = alv[pl.ds(2 * L, L)]
            a3 = alv[pl.ds(3 * L, L)]

            @plsc.parallel_loop(0, N, unroll=2)
            def combine(i):
                out_f[pl.ds(i * L, L)] = (a0 * x0v[i, :] + a1 * s1v[i, :]
                                          + a2 * s2v[i, :] + a3 * s3v[i, :])

            @plsc.parallel_loop(0, EP // L)
            def dot(g):
                rv16 = rowi[pl.ds(g * L, L)] * L
                cv16 = coli[pl.ds(g * L, L)] * L
                acc = zero16
                for d in range(L):
                    acc = acc + (plsc.load_gather(out_f, [rv16 + d])
                                 * plsc.load_gather(out_f, [cv16 + d]))
                part[pl.ds(g * L, L)] = acc

            pltpu.sync_copy(part, shared.at[chunk])

        plsc.subcore_barrier()

        @pl.when(jnp.logical_and(cid == 0, sid == 0))
        def _reduce():
            for t in range(1, NCH):
                pltpu.sync_copy(shared.at[t], tmp)

                @plsc.parallel_loop(0, EP // L, unroll=2)
                def accum(g, t=t):
                    part[pl.ds(g * L, L)] = (part[pl.ds(g * L, L)]
                                             + tmp[pl.ds(g * L, L)])
            pltpu.sync_copy(part, out_hbm)

    return gcn_kernel


_gcn_cache = []


def _gcn(*args):
    if not _gcn_cache:
        _gcn_cache.append(_build())
    return _gcn_cache[0](*args)


def kernel(L_edge_index_, L_self_modules_embedding_parameters_weight_,
           L_self_buffers_alpha_):
    ei = L_edge_index_
    w = L_self_modules_embedding_parameters_weight_
    alpha = L_self_buffers_alpha_
    ei_p = jnp.pad(ei.astype(jnp.int32), ((0, 0), (0, EP - E)))
    wc = w.reshape(N, NCH, L).transpose(1, 0, 2)
    alpha_p = jnp.tile(alpha.astype(jnp.float32)[:, None], (1, L)).reshape(
        4 * L)
    res = _gcn(ei_p, wc, alpha_p)
    return (res[:E],)
